# baseline (speedup 1.0000x reference)
import functools

import jax
import jax.numpy as jnp
from jax.experimental import pallas as pl
from jax.experimental.pallas import tpu as pltpu

_EPS = 1e-5
_NEG_BIG = -1e30
_POS_BIG = 1e30


def _k1_body(nc, ns, x1_ref, x2t_ref, x2_ref, f2_ref, w1_ref, b1_ref, g1_ref,
             msel_ref, ssum_ref, ssq_ref, zcat_ref):
    b = pl.program_id(0)
    i = pl.program_id(1)
    t = msel_ref.shape[1]
    n2 = x2_ref.shape[1]
    c2 = f2_ref.shape[2]

    @pl.when(i == 0)
    def _build_table():
        f2 = f2_ref[0]
        z2v = jnp.dot(f2, w1_ref[:c2, :], preferred_element_type=jnp.float32)
        x2 = x2_ref[0]
        for c in range(3):
            z2v = z2v + x2[:, c:c + 1] * w1_ref[c2 + c:c2 + c + 1, :]
        z2v = z2v + b1_ref[...]
        hi = z2v.astype(jnp.bfloat16)
        lo = (z2v - hi.astype(jnp.float32)).astype(jnp.bfloat16)
        blocks = []
        for c in range(0, n2, 128):
            blocks.append(hi[c:c + 128, :])
            blocks.append(lo[c:c + 128, :])
        zcat_ref[...] = jnp.concatenate(blocks, axis=1)

    x1 = x1_ref[0]
    d = ((x1[:, 0:1] - x2t_ref[0, 0:1, :]) ** 2
         + (x1[:, 1:2] - x2t_ref[0, 1:2, :]) ** 2
         + (x1[:, 2:3] - x2t_ref[0, 2:3, :]) ** 2)

    lane = jax.lax.broadcasted_iota(jnp.int32, (t, 128), 1).astype(jnp.float32)
    zcat = zcat_ref[...]
    cm = zcat.shape[1] // (2 * nc)

    gmax = jnp.full((t, cm), _NEG_BIG, dtype=jnp.float32)
    gmin = jnp.full((t, cm), _POS_BIG, dtype=jnp.float32)
    gsum = jnp.zeros((t, cm), dtype=jnp.float32)
    gsq = jnp.zeros((t, cm), dtype=jnp.float32)

    dl = [d[:, c * 128:(c + 1) * 128] for c in range(nc)]
    for s in range(ns):
        m8 = dl[0]
        ac = jnp.zeros((t, 128), dtype=jnp.float32)
        for c in range(1, nc):
            dc = dl[c]
            pred = dc < m8
            ac = jnp.where(pred, float(c), ac)
            m8 = jnp.where(pred, dc, m8)
        vmin = jnp.min(m8, axis=1, keepdims=True)
        jl = ac * 128.0 + lane
        jmin = jnp.min(jnp.where(m8 == vmin, jl, 2.0 * n2), axis=1,
                       keepdims=True)
        cminf = jnp.floor(jmin * (1.0 / 128.0))
        lminf = jmin - cminf * 128.0
        ohl = lane == lminf
        ohlb = ohl.astype(jnp.bfloat16)
        hop = jnp.dot(ohlb, zcat, preferred_element_type=jnp.float32)
        g = jnp.zeros((t, cm), dtype=jnp.float32)
        for c in range(nc):
            mc = cminf == float(c)
            if s != ns - 1:
                dl[c] = jnp.where(jnp.logical_and(ohl, mc), _POS_BIG, dl[c])
            gc = (hop[:, 2 * c * cm:(2 * c + 1) * cm]
                  + hop[:, (2 * c + 1) * cm:(2 * c + 2) * cm])
            g = g + mc.astype(jnp.float32) * gc
        gmax = jnp.maximum(gmax, g)
        gmin = jnp.minimum(gmin, g)
        gsum = gsum + g
        gsq = gsq + g * g

    q = (x1[:, 0:1] * w1_ref[c2:c2 + 1, :]
         + x1[:, 1:2] * w1_ref[c2 + 1:c2 + 2, :]
         + x1[:, 2:3] * w1_ref[c2 + 2:c2 + 3, :])

    msel_ref[0] = jnp.where(g1_ref[...] >= 0, gmax, gmin) - q

    fns = float(ns)
    part_s = jnp.sum(gsum - fns * q, axis=0, keepdims=True)
    part_q = jnp.sum(gsq - 2.0 * q * gsum + fns * q * q, axis=0,
                     keepdims=True)

    @pl.when(jnp.logical_and(b == 0, i == 0))
    def _init_acc():
        ssum_ref[...] = part_s
        ssq_ref[...] = part_q

    @pl.when(jnp.logical_not(jnp.logical_and(b == 0, i == 0)))
    def _add_acc():
        ssum_ref[...] = ssum_ref[...] + part_s
        ssq_ref[...] = ssq_ref[...] + part_q


def _k2_body(ntot, msel_ref, f1_ref, ssum_ref, ssq_ref, g1_ref, be1_ref,
             w2_ref, b2_ref, y2_ref, s2_ref, sq2_ref):
    b = pl.program_id(0)
    i = pl.program_id(1)
    c1 = f1_ref.shape[2]
    inv_n = 1.0 / float(ntot)
    mean1 = ssum_ref[...] * inv_n
    var1 = ssq_ref[...] * inv_n - mean1 * mean1
    inv1 = jax.lax.rsqrt(var1 + _EPS)
    h = jnp.maximum((msel_ref[0] - mean1) * inv1 * g1_ref[...] + be1_ref[...],
                    0.0)
    nmid = w2_ref.shape[0] - c1
    y2 = (jnp.dot(h, w2_ref[:nmid, :], preferred_element_type=jnp.float32)
          + jnp.dot(f1_ref[0], w2_ref[nmid:, :],
                    preferred_element_type=jnp.float32)
          + b2_ref[...])
    y2_ref[0] = y2
    part_s = jnp.sum(y2, axis=0, keepdims=True)
    part_q = jnp.sum(y2 * y2, axis=0, keepdims=True)

    @pl.when(jnp.logical_and(b == 0, i == 0))
    def _init_acc():
        s2_ref[...] = part_s
        sq2_ref[...] = part_q

    @pl.when(jnp.logical_not(jnp.logical_and(b == 0, i == 0)))
    def _add_acc():
        s2_ref[...] = s2_ref[...] + part_s
        sq2_ref[...] = sq2_ref[...] + part_q


def _k3_body(ntot, y2_ref, s2_ref, sq2_ref, g2_ref, be2_ref, out_ref):
    inv_n = 1.0 / float(ntot)
    mean2 = s2_ref[...] * inv_n
    var2 = sq2_ref[...] * inv_n - mean2 * mean2
    inv2 = jax.lax.rsqrt(var2 + _EPS)
    out_ref[0] = jnp.maximum(
        (y2_ref[0] - mean2) * inv2 * g2_ref[...] + be2_ref[...], 0.0)


@jax.jit
def kernel(xyz1, xyz2, feat1, feat2, W1, b1, g1, be1, W2, b2, g2, be2):
    bsz, n1, _ = xyz1.shape
    n2 = xyz2.shape[1]
    c1 = feat1.shape[2]
    c2 = feat2.shape[2]
    cmid = W1.shape[1]
    cout = W2.shape[1]
    ns = 8
    t = min(512, n1)
    nt = n1 // t
    nc = n2 // 128

    f32 = jnp.float32
    xyz2t = jnp.transpose(xyz2, (0, 2, 1))
    xyz2t = jnp.pad(xyz2t, ((0, 0), (0, 5), (0, 0)))
    b1r = b1.reshape(1, cmid)
    g1r = g1.reshape(1, cmid)
    be1r = be1.reshape(1, cmid)
    b2r = b2.reshape(1, cout)
    g2r = g2.reshape(1, cout)
    be2r = be2.reshape(1, cout)

    msel, ssum, ssq = pl.pallas_call(
        functools.partial(_k1_body, nc, ns),
        grid=(bsz, nt),
        in_specs=[
            pl.BlockSpec((1, t, 3), lambda b, i: (b, i, 0)),
            pl.BlockSpec((1, 8, n2), lambda b, i: (b, 0, 0)),
            pl.BlockSpec((1, n2, 3), lambda b, i: (b, 0, 0)),
            pl.BlockSpec((1, n2, c2), lambda b, i: (b, 0, 0)),
            pl.BlockSpec((c2 + 3, cmid), lambda b, i: (0, 0)),
            pl.BlockSpec((1, cmid), lambda b, i: (0, 0)),
            pl.BlockSpec((1, cmid), lambda b, i: (0, 0)),
        ],
        out_specs=[
            pl.BlockSpec((1, t, cmid), lambda b, i: (b, i, 0)),
            pl.BlockSpec((1, cmid), lambda b, i: (0, 0)),
            pl.BlockSpec((1, cmid), lambda b, i: (0, 0)),
        ],
        out_shape=[
            jax.ShapeDtypeStruct((bsz, n1, cmid), f32),
            jax.ShapeDtypeStruct((1, cmid), f32),
            jax.ShapeDtypeStruct((1, cmid), f32),
        ],
        scratch_shapes=[
            pltpu.VMEM((128, 2 * nc * cmid), jnp.bfloat16),
        ],
    )(xyz1, xyz2t, xyz2, feat2, W1, b1r, g1r)

    ntot1 = bsz * n1 * ns
    y2, s2, sq2 = pl.pallas_call(
        functools.partial(_k2_body, ntot1),
        grid=(bsz, nt),
        in_specs=[
            pl.BlockSpec((1, t, cmid), lambda b, i: (b, i, 0)),
            pl.BlockSpec((1, t, c1), lambda b, i: (b, i, 0)),
            pl.BlockSpec((1, cmid), lambda b, i: (0, 0)),
            pl.BlockSpec((1, cmid), lambda b, i: (0, 0)),
            pl.BlockSpec((1, cmid), lambda b, i: (0, 0)),
            pl.BlockSpec((1, cmid), lambda b, i: (0, 0)),
            pl.BlockSpec((cmid + c1, cout), lambda b, i: (0, 0)),
            pl.BlockSpec((1, cout), lambda b, i: (0, 0)),
        ],
        out_specs=[
            pl.BlockSpec((1, t, cout), lambda b, i: (b, i, 0)),
            pl.BlockSpec((1, cout), lambda b, i: (0, 0)),
            pl.BlockSpec((1, cout), lambda b, i: (0, 0)),
        ],
        out_shape=[
            jax.ShapeDtypeStruct((bsz, n1, cout), f32),
            jax.ShapeDtypeStruct((1, cout), f32),
            jax.ShapeDtypeStruct((1, cout), f32),
        ],
    )(msel, feat1, ssum, ssq, g1r, be1r, W2, b2r)

    ntot2 = bsz * n1
    out = pl.pallas_call(
        functools.partial(_k3_body, ntot2),
        grid=(bsz, nt),
        in_specs=[
            pl.BlockSpec((1, t, cout), lambda b, i: (b, i, 0)),
            pl.BlockSpec((1, cout), lambda b, i: (0, 0)),
            pl.BlockSpec((1, cout), lambda b, i: (0, 0)),
            pl.BlockSpec((1, cout), lambda b, i: (0, 0)),
            pl.BlockSpec((1, cout), lambda b, i: (0, 0)),
        ],
        out_specs=pl.BlockSpec((1, t, cout), lambda b, i: (b, i, 0)),
        out_shape=jax.ShapeDtypeStruct((bsz, n1, cout), f32),
    )(y2, s2, sq2, g2r, be2r)
    return out


# speedup vs baseline: 1.8079x; 1.8079x over previous
import functools

import jax
import jax.numpy as jnp
from jax.experimental import pallas as pl
from jax.experimental.pallas import tpu as pltpu

_EPS = 1e-5
_NEG_BIG = -1e30
_POS_BIG = 1e30


def _k1_body(nc, ns, x1_ref, x2t_ref, x2_ref, f2_ref, w1_ref, b1_ref, g1_ref,
             msel_ref, ssum_ref, ssq_ref, zcat_ref):
    b = pl.program_id(0)
    i = pl.program_id(1)
    t = msel_ref.shape[1]
    n2 = x2_ref.shape[1]
    c2 = f2_ref.shape[2]

    @pl.when(i == 0)
    def _build_table():
        f2 = f2_ref[0]
        z2v = jnp.dot(f2, w1_ref[:c2, :], preferred_element_type=jnp.float32)
        x2 = x2_ref[0]
        for c in range(3):
            z2v = z2v + x2[:, c:c + 1] * w1_ref[c2 + c:c2 + c + 1, :]
        z2v = z2v + b1_ref[...]
        hi = z2v.astype(jnp.bfloat16)
        lo = (z2v - hi.astype(jnp.float32)).astype(jnp.bfloat16)
        zcat_ref[...] = jnp.concatenate([hi, lo], axis=1)

    x1 = x1_ref[0]
    d = ((x1[:, 0:1] - x2t_ref[0, 0:1, :]) ** 2
         + (x1[:, 1:2] - x2t_ref[0, 1:2, :]) ** 2
         + (x1[:, 2:3] - x2t_ref[0, 2:3, :]) ** 2)

    lane = jax.lax.broadcasted_iota(jnp.int32, (t, 128), 1).astype(jnp.float32)
    jall = jax.lax.broadcasted_iota(jnp.int32, (t, n2), 1).astype(jnp.float32)
    zcat = zcat_ref[...]
    cm = zcat.shape[1] // 2

    gmax = jnp.full((t, cm), _NEG_BIG, dtype=jnp.float32)
    gmin = jnp.full((t, cm), _POS_BIG, dtype=jnp.float32)
    gsum = jnp.zeros((t, cm), dtype=jnp.float32)
    gsq = jnp.zeros((t, cm), dtype=jnp.float32)

    for s in range(ns):
        m8 = d[:, 0:128]
        ac = jnp.zeros((t, 128), dtype=jnp.float32)
        for c in range(1, nc):
            dc = d[:, c * 128:(c + 1) * 128]
            pred = dc < m8
            ac = jnp.where(pred, float(c), ac)
            m8 = jnp.where(pred, dc, m8)
        vmin = jnp.min(m8, axis=1, keepdims=True)
        jl = ac * 128.0 + lane
        jmin = jnp.min(jnp.where(m8 == vmin, jl, 2.0 * n2), axis=1,
                       keepdims=True)
        onehot = jall == jmin
        ohb = onehot.astype(jnp.bfloat16)
        if s != ns - 1:
            d = jnp.where(onehot, _POS_BIG, d)
        g2 = jnp.dot(ohb, zcat, preferred_element_type=jnp.float32)
        g = g2[:, :cm] + g2[:, cm:]
        gmax = jnp.maximum(gmax, g)
        gmin = jnp.minimum(gmin, g)
        gsum = gsum + g
        gsq = gsq + g * g

    q = (x1[:, 0:1] * w1_ref[c2:c2 + 1, :]
         + x1[:, 1:2] * w1_ref[c2 + 1:c2 + 2, :]
         + x1[:, 2:3] * w1_ref[c2 + 2:c2 + 3, :])

    msel_ref[0] = jnp.where(g1_ref[...] >= 0, gmax, gmin) - q

    fns = float(ns)
    part_s = jnp.sum(gsum - fns * q, axis=0, keepdims=True)
    part_q = jnp.sum(gsq - 2.0 * q * gsum + fns * q * q, axis=0,
                     keepdims=True)

    @pl.when(jnp.logical_and(b == 0, i == 0))
    def _init_acc():
        ssum_ref[...] = part_s
        ssq_ref[...] = part_q

    @pl.when(jnp.logical_not(jnp.logical_and(b == 0, i == 0)))
    def _add_acc():
        ssum_ref[...] = ssum_ref[...] + part_s
        ssq_ref[...] = ssq_ref[...] + part_q


def _k2_body(ntot, msel_ref, f1_ref, ssum_ref, ssq_ref, g1_ref, be1_ref,
             w2_ref, b2_ref, y2_ref, s2_ref, sq2_ref):
    b = pl.program_id(0)
    i = pl.program_id(1)
    c1 = f1_ref.shape[2]
    inv_n = 1.0 / float(ntot)
    mean1 = ssum_ref[...] * inv_n
    var1 = ssq_ref[...] * inv_n - mean1 * mean1
    inv1 = jax.lax.rsqrt(var1 + _EPS)
    h = jnp.maximum((msel_ref[0] - mean1) * inv1 * g1_ref[...] + be1_ref[...],
                    0.0)
    nmid = w2_ref.shape[0] - c1
    y2 = (jnp.dot(h, w2_ref[:nmid, :], preferred_element_type=jnp.float32)
          + jnp.dot(f1_ref[0], w2_ref[nmid:, :],
                    preferred_element_type=jnp.float32)
          + b2_ref[...])
    y2_ref[0] = y2
    part_s = jnp.sum(y2, axis=0, keepdims=True)
    part_q = jnp.sum(y2 * y2, axis=0, keepdims=True)

    @pl.when(jnp.logical_and(b == 0, i == 0))
    def _init_acc():
        s2_ref[...] = part_s
        sq2_ref[...] = part_q

    @pl.when(jnp.logical_not(jnp.logical_and(b == 0, i == 0)))
    def _add_acc():
        s2_ref[...] = s2_ref[...] + part_s
        sq2_ref[...] = sq2_ref[...] + part_q


def _k3_body(ntot, y2_ref, s2_ref, sq2_ref, g2_ref, be2_ref, out_ref):
    inv_n = 1.0 / float(ntot)
    mean2 = s2_ref[...] * inv_n
    var2 = sq2_ref[...] * inv_n - mean2 * mean2
    inv2 = jax.lax.rsqrt(var2 + _EPS)
    out_ref[0] = jnp.maximum(
        (y2_ref[0] - mean2) * inv2 * g2_ref[...] + be2_ref[...], 0.0)


@jax.jit
def kernel(xyz1, xyz2, feat1, feat2, W1, b1, g1, be1, W2, b2, g2, be2):
    bsz, n1, _ = xyz1.shape
    n2 = xyz2.shape[1]
    c1 = feat1.shape[2]
    c2 = feat2.shape[2]
    cmid = W1.shape[1]
    cout = W2.shape[1]
    ns = 8
    t = min(512, n1)
    nt = n1 // t
    nc = n2 // 128

    f32 = jnp.float32
    xyz2t = jnp.transpose(xyz2, (0, 2, 1))
    xyz2t = jnp.pad(xyz2t, ((0, 0), (0, 5), (0, 0)))
    b1r = b1.reshape(1, cmid)
    g1r = g1.reshape(1, cmid)
    be1r = be1.reshape(1, cmid)
    b2r = b2.reshape(1, cout)
    g2r = g2.reshape(1, cout)
    be2r = be2.reshape(1, cout)

    msel, ssum, ssq = pl.pallas_call(
        functools.partial(_k1_body, nc, ns),
        grid=(bsz, nt),
        in_specs=[
            pl.BlockSpec((1, t, 3), lambda b, i: (b, i, 0)),
            pl.BlockSpec((1, 8, n2), lambda b, i: (b, 0, 0)),
            pl.BlockSpec((1, n2, 3), lambda b, i: (b, 0, 0)),
            pl.BlockSpec((1, n2, c2), lambda b, i: (b, 0, 0)),
            pl.BlockSpec((c2 + 3, cmid), lambda b, i: (0, 0)),
            pl.BlockSpec((1, cmid), lambda b, i: (0, 0)),
            pl.BlockSpec((1, cmid), lambda b, i: (0, 0)),
        ],
        out_specs=[
            pl.BlockSpec((1, t, cmid), lambda b, i: (b, i, 0)),
            pl.BlockSpec((1, cmid), lambda b, i: (0, 0)),
            pl.BlockSpec((1, cmid), lambda b, i: (0, 0)),
        ],
        out_shape=[
            jax.ShapeDtypeStruct((bsz, n1, cmid), f32),
            jax.ShapeDtypeStruct((1, cmid), f32),
            jax.ShapeDtypeStruct((1, cmid), f32),
        ],
        scratch_shapes=[
            pltpu.VMEM((n2, 2 * cmid), jnp.bfloat16),
        ],
    )(xyz1, xyz2t, xyz2, feat2, W1, b1r, g1r)

    ntot1 = bsz * n1 * ns
    y2, s2, sq2 = pl.pallas_call(
        functools.partial(_k2_body, ntot1),
        grid=(bsz, nt),
        in_specs=[
            pl.BlockSpec((1, t, cmid), lambda b, i: (b, i, 0)),
            pl.BlockSpec((1, t, c1), lambda b, i: (b, i, 0)),
            pl.BlockSpec((1, cmid), lambda b, i: (0, 0)),
            pl.BlockSpec((1, cmid), lambda b, i: (0, 0)),
            pl.BlockSpec((1, cmid), lambda b, i: (0, 0)),
            pl.BlockSpec((1, cmid), lambda b, i: (0, 0)),
            pl.BlockSpec((cmid + c1, cout), lambda b, i: (0, 0)),
            pl.BlockSpec((1, cout), lambda b, i: (0, 0)),
        ],
        out_specs=[
            pl.BlockSpec((1, t, cout), lambda b, i: (b, i, 0)),
            pl.BlockSpec((1, cout), lambda b, i: (0, 0)),
            pl.BlockSpec((1, cout), lambda b, i: (0, 0)),
        ],
        out_shape=[
            jax.ShapeDtypeStruct((bsz, n1, cout), f32),
            jax.ShapeDtypeStruct((1, cout), f32),
            jax.ShapeDtypeStruct((1, cout), f32),
        ],
    )(msel, feat1, ssum, ssq, g1r, be1r, W2, b2r)

    ntot2 = bsz * n1
    out = pl.pallas_call(
        functools.partial(_k3_body, ntot2),
        grid=(bsz, nt),
        in_specs=[
            pl.BlockSpec((1, t, cout), lambda b, i: (b, i, 0)),
            pl.BlockSpec((1, cout), lambda b, i: (0, 0)),
            pl.BlockSpec((1, cout), lambda b, i: (0, 0)),
            pl.BlockSpec((1, cout), lambda b, i: (0, 0)),
            pl.BlockSpec((1, cout), lambda b, i: (0, 0)),
        ],
        out_specs=pl.BlockSpec((1, t, cout), lambda b, i: (b, i, 0)),
        out_shape=jax.ShapeDtypeStruct((bsz, n1, cout), f32),
    )(y2, s2, sq2, g2r, be2r)
    return out


# speedup vs baseline: 1.8830x; 1.0415x over previous
import functools

import jax
import jax.numpy as jnp
from jax.experimental import pallas as pl
from jax.experimental.pallas import tpu as pltpu

_EPS = 1e-5
_NEG_BIG = -1e30
_POS_BIG = 1e30


def _k1_body(nc, ns, x1_ref, x2t_ref, x2_ref, f2_ref, w1_ref, b1_ref, g1_ref,
             msel_ref, ssum_ref, ssq_ref, zcat_ref):
    b = pl.program_id(0)
    i = pl.program_id(1)
    t = msel_ref.shape[1]
    n2 = x2_ref.shape[1]
    c2 = f2_ref.shape[2]

    @pl.when(i == 0)
    def _build_table():
        f2 = f2_ref[0]
        z2v = jnp.dot(f2, w1_ref[:c2, :], preferred_element_type=jnp.float32)
        x2 = x2_ref[0]
        for c in range(3):
            z2v = z2v + x2[:, c:c + 1] * w1_ref[c2 + c:c2 + c + 1, :]
        z2v = z2v + b1_ref[...]
        hi = z2v.astype(jnp.bfloat16)
        lo = (z2v - hi.astype(jnp.float32)).astype(jnp.bfloat16)
        zcat_ref[...] = jnp.concatenate([hi, lo], axis=1)

    x1 = x1_ref[0]
    d = ((x1[:, 0:1] - x2t_ref[0, 0:1, :]) ** 2
         + (x1[:, 1:2] - x2t_ref[0, 1:2, :]) ** 2
         + (x1[:, 2:3] - x2t_ref[0, 2:3, :]) ** 2)

    jall = jax.lax.broadcasted_iota(jnp.int32, (t, n2), 1).astype(jnp.float32)
    zcat = zcat_ref[...]
    cm = zcat.shape[1] // 2

    gmax = jnp.full((t, cm), _NEG_BIG, dtype=jnp.float32)
    gmin = jnp.full((t, cm), _POS_BIG, dtype=jnp.float32)
    gsum = jnp.zeros((t, cm), dtype=jnp.float32)
    gsq = jnp.zeros((t, cm), dtype=jnp.float32)

    for s in range(ns):
        m8 = d[:, 0:128]
        for c in range(1, nc):
            m8 = jnp.minimum(m8, d[:, c * 128:(c + 1) * 128])
        vmin = jnp.min(m8, axis=1, keepdims=True)
        jmin = jnp.min(jnp.where(d == vmin, jall, 2.0 * n2), axis=1,
                       keepdims=True)
        onehot = jall == jmin
        ohb = onehot.astype(jnp.bfloat16)
        if s != ns - 1:
            d = jnp.where(onehot, _POS_BIG, d)
        g2 = jnp.dot(ohb, zcat, preferred_element_type=jnp.float32)
        g = g2[:, :cm] + g2[:, cm:]
        gmax = jnp.maximum(gmax, g)
        gmin = jnp.minimum(gmin, g)
        gsum = gsum + g
        gsq = gsq + g * g

    q = (x1[:, 0:1] * w1_ref[c2:c2 + 1, :]
         + x1[:, 1:2] * w1_ref[c2 + 1:c2 + 2, :]
         + x1[:, 2:3] * w1_ref[c2 + 2:c2 + 3, :])

    msel_ref[0] = jnp.where(g1_ref[...] >= 0, gmax, gmin) - q

    fns = float(ns)
    part_s = jnp.sum(gsum - fns * q, axis=0, keepdims=True)
    part_q = jnp.sum(gsq - 2.0 * q * gsum + fns * q * q, axis=0,
                     keepdims=True)

    @pl.when(jnp.logical_and(b == 0, i == 0))
    def _init_acc():
        ssum_ref[...] = part_s
        ssq_ref[...] = part_q

    @pl.when(jnp.logical_not(jnp.logical_and(b == 0, i == 0)))
    def _add_acc():
        ssum_ref[...] = ssum_ref[...] + part_s
        ssq_ref[...] = ssq_ref[...] + part_q


def _k2_body(ntot, msel_ref, f1_ref, ssum_ref, ssq_ref, g1_ref, be1_ref,
             w2_ref, b2_ref, y2_ref, s2_ref, sq2_ref):
    b = pl.program_id(0)
    i = pl.program_id(1)
    c1 = f1_ref.shape[2]
    inv_n = 1.0 / float(ntot)
    mean1 = ssum_ref[...] * inv_n
    var1 = ssq_ref[...] * inv_n - mean1 * mean1
    inv1 = jax.lax.rsqrt(var1 + _EPS)
    h = jnp.maximum((msel_ref[0] - mean1) * inv1 * g1_ref[...] + be1_ref[...],
                    0.0)
    nmid = w2_ref.shape[0] - c1
    y2 = (jnp.dot(h, w2_ref[:nmid, :], preferred_element_type=jnp.float32)
          + jnp.dot(f1_ref[0], w2_ref[nmid:, :],
                    preferred_element_type=jnp.float32)
          + b2_ref[...])
    y2_ref[0] = y2
    part_s = jnp.sum(y2, axis=0, keepdims=True)
    part_q = jnp.sum(y2 * y2, axis=0, keepdims=True)

    @pl.when(jnp.logical_and(b == 0, i == 0))
    def _init_acc():
        s2_ref[...] = part_s
        sq2_ref[...] = part_q

    @pl.when(jnp.logical_not(jnp.logical_and(b == 0, i == 0)))
    def _add_acc():
        s2_ref[...] = s2_ref[...] + part_s
        sq2_ref[...] = sq2_ref[...] + part_q


def _k3_body(ntot, y2_ref, s2_ref, sq2_ref, g2_ref, be2_ref, out_ref):
    inv_n = 1.0 / float(ntot)
    mean2 = s2_ref[...] * inv_n
    var2 = sq2_ref[...] * inv_n - mean2 * mean2
    inv2 = jax.lax.rsqrt(var2 + _EPS)
    out_ref[0] = jnp.maximum(
        (y2_ref[0] - mean2) * inv2 * g2_ref[...] + be2_ref[...], 0.0)


@jax.jit
def kernel(xyz1, xyz2, feat1, feat2, W1, b1, g1, be1, W2, b2, g2, be2):
    bsz, n1, _ = xyz1.shape
    n2 = xyz2.shape[1]
    c1 = feat1.shape[2]
    c2 = feat2.shape[2]
    cmid = W1.shape[1]
    cout = W2.shape[1]
    ns = 8
    t = min(1024, n1)
    nt = n1 // t
    nc = n2 // 128

    f32 = jnp.float32
    xyz2t = jnp.transpose(xyz2, (0, 2, 1))
    xyz2t = jnp.pad(xyz2t, ((0, 0), (0, 5), (0, 0)))
    b1r = b1.reshape(1, cmid)
    g1r = g1.reshape(1, cmid)
    be1r = be1.reshape(1, cmid)
    b2r = b2.reshape(1, cout)
    g2r = g2.reshape(1, cout)
    be2r = be2.reshape(1, cout)

    msel, ssum, ssq = pl.pallas_call(
        functools.partial(_k1_body, nc, ns),
        grid=(bsz, nt),
        in_specs=[
            pl.BlockSpec((1, t, 3), lambda b, i: (b, i, 0)),
            pl.BlockSpec((1, 8, n2), lambda b, i: (b, 0, 0)),
            pl.BlockSpec((1, n2, 3), lambda b, i: (b, 0, 0)),
            pl.BlockSpec((1, n2, c2), lambda b, i: (b, 0, 0)),
            pl.BlockSpec((c2 + 3, cmid), lambda b, i: (0, 0)),
            pl.BlockSpec((1, cmid), lambda b, i: (0, 0)),
            pl.BlockSpec((1, cmid), lambda b, i: (0, 0)),
        ],
        out_specs=[
            pl.BlockSpec((1, t, cmid), lambda b, i: (b, i, 0)),
            pl.BlockSpec((1, cmid), lambda b, i: (0, 0)),
            pl.BlockSpec((1, cmid), lambda b, i: (0, 0)),
        ],
        out_shape=[
            jax.ShapeDtypeStruct((bsz, n1, cmid), f32),
            jax.ShapeDtypeStruct((1, cmid), f32),
            jax.ShapeDtypeStruct((1, cmid), f32),
        ],
        scratch_shapes=[
            pltpu.VMEM((n2, 2 * cmid), jnp.bfloat16),
        ],
    )(xyz1, xyz2t, xyz2, feat2, W1, b1r, g1r)

    ntot1 = bsz * n1 * ns
    y2, s2, sq2 = pl.pallas_call(
        functools.partial(_k2_body, ntot1),
        grid=(bsz, nt),
        in_specs=[
            pl.BlockSpec((1, t, cmid), lambda b, i: (b, i, 0)),
            pl.BlockSpec((1, t, c1), lambda b, i: (b, i, 0)),
            pl.BlockSpec((1, cmid), lambda b, i: (0, 0)),
            pl.BlockSpec((1, cmid), lambda b, i: (0, 0)),
            pl.BlockSpec((1, cmid), lambda b, i: (0, 0)),
            pl.BlockSpec((1, cmid), lambda b, i: (0, 0)),
            pl.BlockSpec((cmid + c1, cout), lambda b, i: (0, 0)),
            pl.BlockSpec((1, cout), lambda b, i: (0, 0)),
        ],
        out_specs=[
            pl.BlockSpec((1, t, cout), lambda b, i: (b, i, 0)),
            pl.BlockSpec((1, cout), lambda b, i: (0, 0)),
            pl.BlockSpec((1, cout), lambda b, i: (0, 0)),
        ],
        out_shape=[
            jax.ShapeDtypeStruct((bsz, n1, cout), f32),
            jax.ShapeDtypeStruct((1, cout), f32),
            jax.ShapeDtypeStruct((1, cout), f32),
        ],
    )(msel, feat1, ssum, ssq, g1r, be1r, W2, b2r)

    ntot2 = bsz * n1
    out = pl.pallas_call(
        functools.partial(_k3_body, ntot2),
        grid=(bsz, nt),
        in_specs=[
            pl.BlockSpec((1, t, cout), lambda b, i: (b, i, 0)),
            pl.BlockSpec((1, cout), lambda b, i: (0, 0)),
            pl.BlockSpec((1, cout), lambda b, i: (0, 0)),
            pl.BlockSpec((1, cout), lambda b, i: (0, 0)),
            pl.BlockSpec((1, cout), lambda b, i: (0, 0)),
        ],
        out_specs=pl.BlockSpec((1, t, cout), lambda b, i: (b, i, 0)),
        out_shape=jax.ShapeDtypeStruct((bsz, n1, cout), f32),
    )(y2, s2, sq2, g2r, be2r)
    return out
